# Initial kernel scaffold; baseline (speedup 1.0000x reference)
import jax
import jax.numpy as jnp
from jax.experimental import pallas as pl


def kernel(tokens, token_table, position_embedding):
    raise NotImplementedError("write your pallas kernel here")


# speedup vs baseline: 3.5905x; 3.5905x over previous
import functools

import jax
import jax.numpy as jnp
from jax import lax
from jax.experimental import pallas as pl
from jax.experimental.pallas import tpu as pltpu
from jax.experimental.pallas import tpu_sc as plsc

_NC = 2
_NS = 16
_NW = _NC * _NS
_L = 16


@functools.partial(jax.jit, static_argnames=("chunk",))
def _embedding_lookup(flat_tokens, token_table, position_embedding, chunk):
    total = flat_tokens.shape[0]
    V, D = token_table.shape
    T = position_embedding.shape[0]
    rows_per_w = total // _NW
    nchunks = rows_per_w // chunk

    mesh = plsc.VectorSubcoreMesh(core_axis_name="c", subcore_axis_name="s")

    @functools.partial(
        pl.kernel,
        mesh=mesh,
        compiler_params=pltpu.CompilerParams(use_tc_tiling_on_sc=False),
        out_type=jax.ShapeDtypeStruct((total, D), jnp.float32),
        scratch_types=[
            pltpu.VMEM((rows_per_w,), jnp.int32),
            pltpu.VMEM((chunk, D), jnp.float32),
            pltpu.VMEM((chunk, D), jnp.float32),
            pltpu.VMEM((T, D), jnp.float32),
            pltpu.SemaphoreType.DMA,
            pltpu.SemaphoreType.DMA,
            pltpu.SemaphoreType.DMA,
            pltpu.SemaphoreType.DMA,
        ],
    )
    def emb_kernel(tok_hbm, tab_hbm, pos_hbm, out_hbm,
                   idx_v, buf0, buf1, pos_v, g_sem0, g_sem1, w_sem0, w_sem1):
        wid = lax.axis_index("s") * _NC + lax.axis_index("c")
        base = wid * rows_per_w
        pltpu.sync_copy(tok_hbm.at[pl.ds(base, rows_per_w)], idx_v)
        pltpu.sync_copy(pos_hbm, pos_v)

        bufs = (buf0, buf1)
        g_sems = (g_sem0, g_sem1)
        w_sems = (w_sem0, w_sem1)
        gcp = [None, None]
        wcp = [None, None]

        gcp[0] = pltpu.async_copy(
            tab_hbm.at[idx_v.at[pl.ds(0, chunk)]], buf0, g_sem0)

        for g in range(nchunks):
            cur = g % 2
            nxt = (g + 1) % 2
            if g + 1 < nchunks:
                if wcp[nxt] is not None:
                    wcp[nxt].wait()
                    wcp[nxt] = None
                gcp[nxt] = pltpu.async_copy(
                    tab_hbm.at[idx_v.at[pl.ds((g + 1) * chunk, chunk)]],
                    bufs[nxt], g_sems[nxt])
            gcp[cur].wait()

            buf = bufs[cur]

            @pl.loop(0, chunk)
            def _add_pos(r):
                t = lax.rem(r, T)
                for c in range(D // _L):
                    p = pos_v[t, pl.ds(c * _L, _L)]
                    plsc.addupdate(buf.at[r, pl.ds(c * _L, _L)], p)

            wcp[cur] = pltpu.async_copy(
                buf, out_hbm.at[pl.ds(base + g * chunk, chunk)], w_sems[cur])

        for cur in range(2):
            if wcp[cur] is not None:
                wcp[cur].wait()

    return emb_kernel(flat_tokens, token_table, position_embedding)


def kernel(tokens, token_table, position_embedding):
    B, T = tokens.shape
    D = token_table.shape[1]
    flat_tokens = tokens.reshape(B * T).astype(jnp.int32)
    out = _embedding_lookup(flat_tokens, token_table, position_embedding,
                            chunk=400)
    return out.reshape(B, T, D)


# speedup vs baseline: 4.4982x; 1.2528x over previous
import functools

import jax
import jax.numpy as jnp
from jax import lax
from jax.experimental import pallas as pl
from jax.experimental.pallas import tpu as pltpu
from jax.experimental.pallas import tpu_sc as plsc

_NC = 2
_NS = 16
_NW = _NC * _NS
_L = 16


@functools.partial(jax.jit, static_argnames=("chunk",))
def _embedding_lookup(flat_tokens, token_table, position_embedding, chunk):
    total = flat_tokens.shape[0]
    V, D = token_table.shape
    T = position_embedding.shape[0]
    rows_per_w = total // _NW
    nchunks = rows_per_w // chunk

    mesh = plsc.VectorSubcoreMesh(core_axis_name="c", subcore_axis_name="s")

    @functools.partial(
        pl.kernel,
        mesh=mesh,
        compiler_params=pltpu.CompilerParams(use_tc_tiling_on_sc=False),
        out_type=jax.ShapeDtypeStruct((total, D), jnp.float32),
        scratch_types=[
            pltpu.VMEM((rows_per_w,), jnp.int32),
            pltpu.VMEM((chunk, D), jnp.float32),
            pltpu.VMEM((chunk, D), jnp.float32),
            pltpu.VMEM((chunk, D), jnp.float32),
            pltpu.SemaphoreType.DMA,
            pltpu.SemaphoreType.DMA,
            pltpu.SemaphoreType.DMA,
            pltpu.SemaphoreType.DMA,
        ],
    )
    def emb_kernel(tok_hbm, tab_hbm, pos_hbm, out_hbm,
                   idx_v, buf0, buf1, pos_rep, g_sem0, g_sem1, w_sem0, w_sem1):
        wid = lax.axis_index("s") * _NC + lax.axis_index("c")
        base = wid * rows_per_w
        pltpu.sync_copy(tok_hbm.at[pl.ds(base, rows_per_w)], idx_v)
        for k in range(chunk // T):
            pltpu.sync_copy(pos_hbm, pos_rep.at[pl.ds(k * T, T)])

        bufs = (buf0, buf1)
        g_sems = (g_sem0, g_sem1)
        w_sems = (w_sem0, w_sem1)
        gcp = [None, None]
        wcp = [None, None]

        def start_gather(g, slot):
            return pltpu.async_copy(
                tab_hbm.at[idx_v.at[pl.ds(g * chunk, chunk)]],
                bufs[slot], g_sems[slot])

        gcp[0] = start_gather(0, 0)

        for g in range(nchunks):
            cur = g % 2
            nxt = (g + 1) % 2
            if g + 1 < nchunks:
                if wcp[nxt] is not None:
                    wcp[nxt].wait()
                    wcp[nxt] = None
                gcp[nxt] = start_gather(g + 1, nxt)
            gcp[cur].wait()

            buf = bufs[cur]

            @functools.partial(plsc.parallel_loop, 0, chunk, unroll=4)
            def _add_pos(r):
                for c in range(D // _L):
                    p = pos_rep[r, pl.ds(c * _L, _L)]
                    plsc.addupdate(buf.at[r, pl.ds(c * _L, _L)], p)

            wcp[cur] = pltpu.async_copy(
                bufs[cur], out_hbm.at[pl.ds(base + g * chunk, chunk)],
                w_sems[cur])

        for cur in range(2):
            if wcp[cur] is not None:
                wcp[cur].wait()

    return emb_kernel(flat_tokens, token_table, position_embedding)


def kernel(tokens, token_table, position_embedding):
    B, T = tokens.shape
    D = token_table.shape[1]
    flat_tokens = tokens.reshape(B * T).astype(jnp.int32)
    out = _embedding_lookup(flat_tokens, token_table, position_embedding,
                            chunk=400)
    return out.reshape(B, T, D)
